# baseline (speedup 1.0000x reference)
import functools

import jax
import jax.numpy as jnp
from jax.experimental import pallas as pl

NUM_XYZ_FREQ = 10
NUM_DIR_FREQ = 4
NUM_EXPERTS = 8
HIDDEN = 256
RGB_HIDDEN = 128
TEMPERATURE = 0.166667
TILE_N = 256


def _pe(v, degree):
    y = jnp.concatenate([(2.0 ** i) * v for i in range(degree)], -1)
    return jnp.concatenate([v, jnp.sin(y), jnp.cos(y)], -1)


def _fused_kernel(x_ref, gum_ref, w0_ref, b0_ref, w1_ref, b1_ref,
                  ws_all_ref, bs_all_ref, wsig_t_ref, bsig_ref,
                  wr1a_ref, wr1b_all_ref, br1_all_ref, wr2bd_ref, br2_all_ref,
                  out_ref):
    x = x_ref[...]
    xyz_pe = _pe(x[:, :3], NUM_XYZ_FREQ)
    vd_pe = _pe(x[:, 3:6], NUM_DIR_FREQ)

    h0 = jnp.maximum(jnp.dot(xyz_pe, w0_ref[...]) + b0_ref[...], 0.0)
    y = jnp.maximum(jnp.dot(h0, w1_ref[...]) + b1_ref[...], 0.0)

    so_all = jnp.maximum(jnp.dot(y, ws_all_ref[...]) + bs_all_ref[...], 0.0)

    prod = so_all * wsig_t_ref[...]
    col = jax.lax.broadcasted_iota(jnp.int32, (HIDDEN * NUM_EXPERTS, NUM_EXPERTS), 0)
    exp = jax.lax.broadcasted_iota(jnp.int32, (HIDDEN * NUM_EXPERTS, NUM_EXPERTS), 1)
    mask = ((col // HIDDEN) == exp).astype(jnp.float32)
    sig_lin = jnp.dot(prod, mask) + bsig_ref[...]
    sigmas = jax.nn.softplus(sig_lin)

    log_logits = jax.nn.log_softmax(jnp.log(sigmas + 1e-10) / TEMPERATURE, axis=-1)
    gates_soft = jax.nn.softmax(log_logits + gum_ref[...], axis=-1)
    index = jnp.argmax(gates_soft, axis=-1)
    lane = jax.lax.broadcasted_iota(jnp.int32, gates_soft.shape, 1)
    onehot = (lane == index[:, None]).astype(jnp.float32)

    sigma_pooled = jnp.sum(sigmas * onehot, axis=-1, keepdims=True)

    vdc_all = jnp.dot(vd_pe, wr1b_all_ref[...])
    h_parts = []
    for i in range(NUM_EXPERTS):
        so_i = so_all[:, i * HIDDEN:(i + 1) * HIDDEN]
        h_parts.append(jnp.dot(so_i, wr1a_ref[i]))
    h_all = jnp.maximum(jnp.concatenate(h_parts, axis=-1)
                        + vdc_all + br1_all_ref[...], 0.0)

    rgb24 = jnp.dot(h_all, wr2bd_ref[...]) + br2_all_ref[...]
    rgb = jnp.zeros((x.shape[0], 3), jnp.float32)
    for i in range(NUM_EXPERTS):
        rgb_i = jax.nn.sigmoid(rgb24[:, 3 * i:3 * i + 3])
        rgb = rgb + rgb_i * onehot[:, i:i + 1]

    out_ref[...] = jnp.concatenate([rgb, sigma_pooled], axis=-1)


@jax.jit
def kernel(x, W0, b0, W1, b1, Ws_e, bs_e, Wsig, bsig, Wr1, br1, Wr2, br2, gumbel):
    n = x.shape[0]
    d_e = HIDDEN * NUM_EXPERTS
    d_h = RGB_HIDDEN * NUM_EXPERTS

    ws_all = jnp.transpose(Ws_e, (1, 0, 2)).reshape(HIDDEN, d_e)
    bs_all = bs_e.reshape(1, d_e)
    wsig_t = jnp.tile(Wsig.reshape(1, HIDDEN), (1, NUM_EXPERTS))
    bsig2 = jnp.broadcast_to(bsig.reshape(1, 1), (1, NUM_EXPERTS))
    wr1a = Wr1[:, :HIDDEN, :]
    wr1b_all = jnp.transpose(Wr1[:, HIDDEN:, :], (1, 0, 2)).reshape(-1, d_h)
    br1_all = br1.reshape(1, d_h)
    eyee = jnp.eye(NUM_EXPERTS, dtype=Wr2.dtype)
    wr2bd = (eyee[:, None, :, None] * Wr2[:, :, None, :]).reshape(d_h, 3 * NUM_EXPERTS)
    br2_all = br2.reshape(1, 3 * NUM_EXPERTS)

    grid = (n // TILE_N,)
    full = lambda s: pl.BlockSpec(s, lambda i: (0,) * len(s))
    row = lambda c: pl.BlockSpec((TILE_N, c), lambda i: (i, 0))

    out = pl.pallas_call(
        _fused_kernel,
        grid=grid,
        in_specs=[
            row(6), row(NUM_EXPERTS),
            full(W0.shape), full((1, HIDDEN)), full(W1.shape), full((1, HIDDEN)),
            full((HIDDEN, d_e)), full((1, d_e)), full((1, d_e)), full((1, NUM_EXPERTS)),
            full((NUM_EXPERTS, HIDDEN, RGB_HIDDEN)), full((27, d_h)), full((1, d_h)),
            full((d_h, 3 * NUM_EXPERTS)), full((1, 3 * NUM_EXPERTS)),
        ],
        out_specs=row(4),
        out_shape=jax.ShapeDtypeStruct((n, 4), jnp.float32),
    )(x, gumbel, W0, b0.reshape(1, -1), W1, b1.reshape(1, -1),
      ws_all, bs_all, wsig_t, bsig2,
      wr1a, wr1b_all, br1_all, wr2bd, br2_all)
    return out


# speedup vs baseline: 1.0224x; 1.0224x over previous
import functools

import jax
import jax.numpy as jnp
from jax.experimental import pallas as pl

NUM_XYZ_FREQ = 10
NUM_DIR_FREQ = 4
NUM_EXPERTS = 8
HIDDEN = 256
RGB_HIDDEN = 128
TEMPERATURE = 0.166667
TILE_N = 256


def _pe(v, degree):
    y = jnp.concatenate([(2.0 ** i) * v for i in range(degree)], -1)
    return jnp.concatenate([v, jnp.sin(y), jnp.cos(y)], -1)


def _fused_kernel(x_ref, gum_ref, w0_ref, b0_ref, w1_ref, b1_ref,
                  ws_all_ref, bs_all_ref, wsig_t_ref, bsig_ref,
                  wr1a_ref, wr1b_all_ref, br1_all_ref, wr2bd_ref, br2_all_ref,
                  out_ref):
    bf = jnp.bfloat16
    mm = functools.partial(jnp.dot, preferred_element_type=jnp.float32)
    x = x_ref[...]
    xyz_pe = _pe(x[:, :3], NUM_XYZ_FREQ)
    vd_pe = _pe(x[:, 3:6], NUM_DIR_FREQ)

    h0 = jnp.maximum(mm(xyz_pe.astype(bf), w0_ref[...]) + b0_ref[...], 0.0)
    y = jnp.maximum(mm(h0.astype(bf), w1_ref[...]) + b1_ref[...], 0.0)

    so_all = jnp.maximum(mm(y.astype(bf), ws_all_ref[...]) + bs_all_ref[...], 0.0)

    prod = so_all * wsig_t_ref[...]
    col = jax.lax.broadcasted_iota(jnp.int32, (HIDDEN * NUM_EXPERTS, NUM_EXPERTS), 0)
    exp = jax.lax.broadcasted_iota(jnp.int32, (HIDDEN * NUM_EXPERTS, NUM_EXPERTS), 1)
    mask = ((col // HIDDEN) == exp).astype(bf)
    sig_lin = mm(prod.astype(bf), mask) + bsig_ref[...]
    sigmas = jax.nn.softplus(sig_lin)

    log_logits = jax.nn.log_softmax(jnp.log(sigmas + 1e-10) / TEMPERATURE, axis=-1)
    gates_soft = jax.nn.softmax(log_logits + gum_ref[...], axis=-1)
    index = jnp.argmax(gates_soft, axis=-1)
    lane = jax.lax.broadcasted_iota(jnp.int32, gates_soft.shape, 1)
    onehot = (lane == index[:, None]).astype(jnp.float32)

    sigma_pooled = jnp.sum(sigmas * onehot, axis=-1, keepdims=True)

    vdc_all = mm(vd_pe.astype(bf), wr1b_all_ref[...])
    so_bf = so_all.astype(bf)
    h_parts = []
    for i in range(NUM_EXPERTS):
        so_i = so_bf[:, i * HIDDEN:(i + 1) * HIDDEN]
        h_parts.append(mm(so_i, wr1a_ref[i]))
    h_all = jnp.maximum(jnp.concatenate(h_parts, axis=-1)
                        + vdc_all + br1_all_ref[...], 0.0)

    rgb24 = mm(h_all.astype(bf), wr2bd_ref[...]) + br2_all_ref[...]
    rgb = jnp.zeros((x.shape[0], 3), jnp.float32)
    for i in range(NUM_EXPERTS):
        rgb_i = jax.nn.sigmoid(rgb24[:, 3 * i:3 * i + 3])
        rgb = rgb + rgb_i * onehot[:, i:i + 1]

    out_ref[...] = jnp.concatenate([rgb, sigma_pooled], axis=-1)


@jax.jit
def kernel(x, W0, b0, W1, b1, Ws_e, bs_e, Wsig, bsig, Wr1, br1, Wr2, br2, gumbel):
    n = x.shape[0]
    d_e = HIDDEN * NUM_EXPERTS
    d_h = RGB_HIDDEN * NUM_EXPERTS

    bf = jnp.bfloat16
    ws_all = jnp.transpose(Ws_e, (1, 0, 2)).reshape(HIDDEN, d_e).astype(bf)
    bs_all = bs_e.reshape(1, d_e)
    wsig_t = jnp.tile(Wsig.reshape(1, HIDDEN), (1, NUM_EXPERTS))
    bsig2 = jnp.broadcast_to(bsig.reshape(1, 1), (1, NUM_EXPERTS))
    wr1a = Wr1[:, :HIDDEN, :].astype(bf)
    wr1b_all = jnp.transpose(Wr1[:, HIDDEN:, :], (1, 0, 2)).reshape(-1, d_h).astype(bf)
    br1_all = br1.reshape(1, d_h)
    eyee = jnp.eye(NUM_EXPERTS, dtype=Wr2.dtype)
    wr2bd = (eyee[:, None, :, None] * Wr2[:, :, None, :]).reshape(d_h, 3 * NUM_EXPERTS).astype(bf)
    br2_all = br2.reshape(1, 3 * NUM_EXPERTS)

    grid = (n // TILE_N,)
    full = lambda s: pl.BlockSpec(s, lambda i: (0,) * len(s))
    row = lambda c: pl.BlockSpec((TILE_N, c), lambda i: (i, 0))

    out = pl.pallas_call(
        _fused_kernel,
        grid=grid,
        in_specs=[
            row(6), row(NUM_EXPERTS),
            full(W0.shape), full((1, HIDDEN)), full(W1.shape), full((1, HIDDEN)),
            full((HIDDEN, d_e)), full((1, d_e)), full((1, d_e)), full((1, NUM_EXPERTS)),
            full((NUM_EXPERTS, HIDDEN, RGB_HIDDEN)), full((27, d_h)), full((1, d_h)),
            full((d_h, 3 * NUM_EXPERTS)), full((1, 3 * NUM_EXPERTS)),
        ],
        out_specs=row(4),
        out_shape=jax.ShapeDtypeStruct((n, 4), jnp.float32),
    )(x, gumbel, W0.astype(bf), b0.reshape(1, -1), W1.astype(bf), b1.reshape(1, -1),
      ws_all, bs_all, wsig_t, bsig2,
      wr1a, wr1b_all, br1_all, wr2bd, br2_all)
    return out


# speedup vs baseline: 1.5593x; 1.5251x over previous
import functools

import jax
import jax.numpy as jnp
import numpy as np
from jax.experimental import pallas as pl

NUM_XYZ_FREQ = 10
NUM_DIR_FREQ = 4
NUM_EXPERTS = 8
HIDDEN = 256
RGB_HIDDEN = 128
TEMPERATURE = 0.166667
TILE_N = 512
D_E = HIDDEN * NUM_EXPERTS
D_H = RGB_HIDDEN * NUM_EXPERTS
PE_LANES = 128


def _pe_matrices():
    m = np.zeros((6, PE_LANES), np.float32)
    c = np.zeros((1, PE_LANES), np.float32)
    half_pi = np.float32(np.pi / 2)
    for i in range(NUM_XYZ_FREQ):
        for ax in range(3):
            m[ax, i * 3 + ax] = 2.0 ** i
            m[ax, 30 + i * 3 + ax] = 2.0 ** i
            c[0, 30 + i * 3 + ax] = half_pi
    for i in range(NUM_DIR_FREQ):
        for ax in range(3):
            m[3 + ax, 60 + i * 3 + ax] = 2.0 ** i
            m[3 + ax, 72 + i * 3 + ax] = 2.0 ** i
            c[0, 72 + i * 3 + ax] = half_pi
    return m, c


_PE_M, _PE_C = _pe_matrices()


def _fused_kernel(x_ref, gum_ref, pe_m_ref, pe_c_ref, wx_ref, ws_cat_ref, b0_ref, w1_ref, b1_ref,
                  ws_all_ref, bs_all_ref, wsig_bd_ref, bsig_ref,
                  wr1a_ref, br1_all_ref, wr2bd_ref, br2_all_ref,
                  rep8_ref, sum24_ref, out_ref):
    bf = jnp.bfloat16
    mm = functools.partial(jnp.dot, preferred_element_type=jnp.float32)
    mmb = lambda a, w: jnp.dot(a, w, preferred_element_type=jnp.float32).astype(bf)
    x = x_ref[...]

    z = jax.lax.dot_general(x, pe_m_ref[...],
                            (((1,), (0,)), ((), ())),
                            precision=jax.lax.Precision.HIGHEST) + pe_c_ref[...]
    s = jnp.sin(z).astype(bf)

    pre = mm(s, ws_cat_ref[...]) + mm(x.astype(bf), wx_ref[...])
    h0 = jnp.maximum(pre[:, :HIDDEN] + b0_ref[...], 0.0)
    vdc_all = pre[:, HIDDEN:].astype(bf)

    y = jnp.maximum(mm(h0.astype(bf), w1_ref[...]) + b1_ref[...], 0.0)
    so_all = jnp.maximum(mmb(y.astype(bf), ws_all_ref[...]) + bs_all_ref[...],
                         bf(0.0))

    sig_lin = mm(so_all, wsig_bd_ref[...]) + bsig_ref[...]
    sigmas = jax.nn.softplus(sig_lin)

    score = jnp.log(sigmas + 1e-10) / TEMPERATURE + gum_ref[...]
    index = jnp.argmax(score, axis=-1)
    lane = jax.lax.broadcasted_iota(jnp.int32, score.shape, 1)
    onehot = (lane == index[:, None]).astype(jnp.float32)

    sigma_pooled = jnp.sum(sigmas * onehot, axis=-1, keepdims=True)

    h_parts = [mmb(so_all[:, i * HIDDEN:(i + 1) * HIDDEN], wr1a_ref[i])
               for i in range(NUM_EXPERTS)]
    h_all = jnp.maximum(jnp.concatenate(h_parts, axis=-1) + vdc_all
                        + br1_all_ref[...], bf(0.0))

    rgb24 = jax.nn.sigmoid(mm(h_all, wr2bd_ref[...]) + br2_all_ref[...])
    oh24 = mm(onehot.astype(bf), rep8_ref[...])
    rgb = jax.lax.dot_general(rgb24 * oh24, sum24_ref[...],
                              (((1,), (0,)), ((), ())),
                              precision=jax.lax.Precision.HIGHEST)

    out_ref[...] = jnp.concatenate([rgb, sigma_pooled], axis=-1)


@jax.jit
def kernel(x, W0, b0, W1, b1, Ws_e, bs_e, Wsig, bsig, Wr1, br1, Wr2, br2, gumbel):
    n = x.shape[0]
    bf = jnp.bfloat16

    wr1b_all = jnp.transpose(Wr1[:, HIDDEN:, :], (1, 0, 2)).reshape(27, D_H)
    z256 = jnp.zeros((PE_LANES - 60, HIDDEN), W0.dtype)
    ws_xyz = jnp.concatenate([W0[3:63], z256], 0)
    ws_vd = jnp.concatenate([jnp.zeros((60, D_H), W0.dtype), wr1b_all[3:27],
                             jnp.zeros((PE_LANES - 84, D_H), W0.dtype)], 0)
    ws_cat = jnp.concatenate([ws_xyz, ws_vd], 1).astype(bf)
    wx_xyz = jnp.concatenate([W0[:3], jnp.zeros((3, HIDDEN), W0.dtype)], 0)
    wx_vd = jnp.concatenate([jnp.zeros((3, D_H), W0.dtype), wr1b_all[:3]], 0)
    wx = jnp.concatenate([wx_xyz, wx_vd], 1).astype(bf)

    ws_all = jnp.transpose(Ws_e, (1, 0, 2)).reshape(HIDDEN, D_E).astype(bf)
    bs_all = bs_e.reshape(1, D_E).astype(bf)
    eyee = jnp.eye(NUM_EXPERTS, dtype=W0.dtype)
    wsig_bd = (eyee[:, :, None] * Wsig.reshape(1, 1, HIDDEN)
               ).transpose(0, 2, 1).reshape(D_E, NUM_EXPERTS).astype(bf)
    bsig2 = jnp.broadcast_to(bsig.reshape(1, 1), (1, NUM_EXPERTS))
    wr1a = Wr1[:, :HIDDEN, :].astype(bf)
    br1_all = br1.reshape(1, D_H).astype(bf)
    wr2bd = (eyee[:, None, :, None] * Wr2[:, :, None, :]).reshape(D_H, 24).astype(bf)
    br2_all = br2.reshape(1, 24)
    rep8 = jnp.repeat(jnp.eye(NUM_EXPERTS, dtype=bf), 3, axis=1)
    sum24 = jnp.tile(jnp.eye(3, dtype=jnp.float32), (NUM_EXPERTS, 1))

    grid = (n // TILE_N,)
    full = lambda s: pl.BlockSpec(s, lambda i: (0,) * len(s))
    row = lambda c: pl.BlockSpec((TILE_N, c), lambda i: (i, 0))

    out = pl.pallas_call(
        _fused_kernel,
        grid=grid,
        in_specs=[
            row(6), row(NUM_EXPERTS),
            full((6, PE_LANES)), full((1, PE_LANES)),
            full((6, HIDDEN + D_H)), full((PE_LANES, HIDDEN + D_H)),
            full((1, HIDDEN)), full((HIDDEN, HIDDEN)), full((1, HIDDEN)),
            full((HIDDEN, D_E)), full((1, D_E)), full((D_E, NUM_EXPERTS)),
            full((1, NUM_EXPERTS)),
            full((NUM_EXPERTS, HIDDEN, RGB_HIDDEN)), full((1, D_H)),
            full((D_H, 24)), full((1, 24)),
            full((NUM_EXPERTS, 24)), full((24, 3)),
        ],
        out_specs=row(4),
        out_shape=jax.ShapeDtypeStruct((n, 4), jnp.float32),
    )(x, gumbel, jnp.asarray(_PE_M), jnp.asarray(_PE_C),
      wx, ws_cat, b0.reshape(1, -1), W1.astype(bf), b1.reshape(1, -1),
      ws_all, bs_all, wsig_bd, bsig2,
      wr1a, br1_all, wr2bd, br2_all, rep8, sum24)
    return out


# speedup vs baseline: 1.7769x; 1.1396x over previous
import functools

import jax
import jax.numpy as jnp
import numpy as np
from jax.experimental import pallas as pl

NUM_XYZ_FREQ = 10
NUM_DIR_FREQ = 4
NUM_EXPERTS = 8
HIDDEN = 256
RGB_HIDDEN = 128
TEMPERATURE = 0.166667
TILE_N = 512
D_E = HIDDEN * NUM_EXPERTS
D_H = RGB_HIDDEN * NUM_EXPERTS
PE_LANES = 128


def _pe_matrices():
    m = np.zeros((6, PE_LANES), np.float32)
    c = np.zeros((1, PE_LANES), np.float32)
    half_pi = np.float32(np.pi / 2)
    for i in range(NUM_XYZ_FREQ):
        for ax in range(3):
            m[ax, i * 3 + ax] = 2.0 ** i
            m[ax, 30 + i * 3 + ax] = 2.0 ** i
            c[0, 30 + i * 3 + ax] = half_pi
    for i in range(NUM_DIR_FREQ):
        for ax in range(3):
            m[3 + ax, 60 + i * 3 + ax] = 2.0 ** i
            m[3 + ax, 72 + i * 3 + ax] = 2.0 ** i
            c[0, 72 + i * 3 + ax] = half_pi
    return m, c


_PE_M, _PE_C = _pe_matrices()

_S1, _S2, _S3, _S4, _S5 = (9.99984593e-01, -1.66632594e-01, 8.31238828e-03,
                           -1.93162699e-04, 2.17325696e-06)
_INV2PI = 0.15915494309189535
_RC1 = 6.28125
_RC2 = 2.0 * np.pi - 6.28125


def _cheap_sin(z):
    k = jnp.round(z * _INV2PI)
    r = (z - k * _RC1) - k * _RC2
    u = r * r
    return r * (_S1 + u * (_S2 + u * (_S3 + u * (_S4 + u * _S5))))


def _fused_kernel(x_ref, gum_ref, pe_m_ref, pe_c_ref, wx_ref, ws_cat_ref, b0_ref, w1_ref, b1_ref,
                  ws_all_ref, bs_all_ref, wsig_bd_ref, bsig_ref,
                  wr1a_ref, br1_all_ref, wr2bd_ref, br2_all_ref,
                  rep8_ref, sum24_ref, out_ref):
    bf = jnp.bfloat16
    mm = functools.partial(jnp.dot, preferred_element_type=jnp.float32)
    mmb = lambda a, w: jnp.dot(a, w, preferred_element_type=jnp.float32).astype(bf)
    x = x_ref[...]

    z = jax.lax.dot_general(x, pe_m_ref[...],
                            (((1,), (0,)), ((), ())),
                            precision=jax.lax.Precision.HIGHEST) + pe_c_ref[...]
    s = _cheap_sin(z).astype(bf)

    pre = mm(s, ws_cat_ref[...]) + mm(x.astype(bf), wx_ref[...])
    h0 = jnp.maximum(pre[:, :HIDDEN] + b0_ref[...], 0.0)
    vdc_all = pre[:, HIDDEN:].astype(bf)

    y = jnp.maximum(mm(h0.astype(bf), w1_ref[...]) + b1_ref[...], 0.0)
    so_all = jnp.maximum(mmb(y.astype(bf), ws_all_ref[...]) + bs_all_ref[...],
                         bf(0.0))

    sig_lin = mm(so_all, wsig_bd_ref[...]) + bsig_ref[...]
    sigmas = jax.nn.softplus(sig_lin)

    score = jnp.log(sigmas + 1e-10) / TEMPERATURE + gum_ref[...]
    index = jnp.argmax(score, axis=-1)
    lane = jax.lax.broadcasted_iota(jnp.int32, score.shape, 1)
    onehot = (lane == index[:, None]).astype(jnp.float32)

    sigma_pooled = jnp.sum(sigmas * onehot, axis=-1, keepdims=True)

    h_parts = [mmb(so_all[:, i * HIDDEN:(i + 1) * HIDDEN], wr1a_ref[i])
               for i in range(NUM_EXPERTS)]
    h_all = jnp.maximum(jnp.concatenate(h_parts, axis=-1) + vdc_all
                        + br1_all_ref[...], bf(0.0))

    rgb24 = jax.nn.sigmoid(mm(h_all, wr2bd_ref[...]) + br2_all_ref[...])
    oh24 = mm(onehot.astype(bf), rep8_ref[...])
    rgb = jax.lax.dot_general(rgb24 * oh24, sum24_ref[...],
                              (((1,), (0,)), ((), ())),
                              precision=jax.lax.Precision.HIGHEST)

    out_ref[...] = jnp.concatenate([rgb, sigma_pooled], axis=-1)


@jax.jit
def kernel(x, W0, b0, W1, b1, Ws_e, bs_e, Wsig, bsig, Wr1, br1, Wr2, br2, gumbel):
    n = x.shape[0]
    bf = jnp.bfloat16

    wr1b_all = jnp.transpose(Wr1[:, HIDDEN:, :], (1, 0, 2)).reshape(27, D_H)
    z256 = jnp.zeros((PE_LANES - 60, HIDDEN), W0.dtype)
    ws_xyz = jnp.concatenate([W0[3:63], z256], 0)
    ws_vd = jnp.concatenate([jnp.zeros((60, D_H), W0.dtype), wr1b_all[3:27],
                             jnp.zeros((PE_LANES - 84, D_H), W0.dtype)], 0)
    ws_cat = jnp.concatenate([ws_xyz, ws_vd], 1).astype(bf)
    wx_xyz = jnp.concatenate([W0[:3], jnp.zeros((3, HIDDEN), W0.dtype)], 0)
    wx_vd = jnp.concatenate([jnp.zeros((3, D_H), W0.dtype), wr1b_all[:3]], 0)
    wx = jnp.concatenate([wx_xyz, wx_vd], 1).astype(bf)

    ws_all = jnp.transpose(Ws_e, (1, 0, 2)).reshape(HIDDEN, D_E).astype(bf)
    bs_all = bs_e.reshape(1, D_E).astype(bf)
    eyee = jnp.eye(NUM_EXPERTS, dtype=W0.dtype)
    wsig_bd = (eyee[:, :, None] * Wsig.reshape(1, 1, HIDDEN)
               ).transpose(0, 2, 1).reshape(D_E, NUM_EXPERTS).astype(bf)
    bsig2 = jnp.broadcast_to(bsig.reshape(1, 1), (1, NUM_EXPERTS))
    wr1a = Wr1[:, :HIDDEN, :].astype(bf)
    br1_all = br1.reshape(1, D_H).astype(bf)
    wr2bd = (eyee[:, None, :, None] * Wr2[:, :, None, :]).reshape(D_H, 24).astype(bf)
    br2_all = br2.reshape(1, 24)
    rep8 = jnp.repeat(jnp.eye(NUM_EXPERTS, dtype=bf), 3, axis=1)
    sum24 = jnp.tile(jnp.eye(3, dtype=jnp.float32), (NUM_EXPERTS, 1))

    grid = (n // TILE_N,)
    full = lambda s: pl.BlockSpec(s, lambda i: (0,) * len(s))
    row = lambda c: pl.BlockSpec((TILE_N, c), lambda i: (i, 0))

    out = pl.pallas_call(
        _fused_kernel,
        grid=grid,
        in_specs=[
            row(6), row(NUM_EXPERTS),
            full((6, PE_LANES)), full((1, PE_LANES)),
            full((6, HIDDEN + D_H)), full((PE_LANES, HIDDEN + D_H)),
            full((1, HIDDEN)), full((HIDDEN, HIDDEN)), full((1, HIDDEN)),
            full((HIDDEN, D_E)), full((1, D_E)), full((D_E, NUM_EXPERTS)),
            full((1, NUM_EXPERTS)),
            full((NUM_EXPERTS, HIDDEN, RGB_HIDDEN)), full((1, D_H)),
            full((D_H, 24)), full((1, 24)),
            full((NUM_EXPERTS, 24)), full((24, 3)),
        ],
        out_specs=row(4),
        out_shape=jax.ShapeDtypeStruct((n, 4), jnp.float32),
    )(x, gumbel, jnp.asarray(_PE_M), jnp.asarray(_PE_C),
      wx, ws_cat, b0.reshape(1, -1), W1.astype(bf), b1.reshape(1, -1),
      ws_all, bs_all, wsig_bd, bsig2,
      wr1a, br1_all, wr2bd, br2_all, rep8, sum24)
    return out
